# baseline (speedup 1.0000x reference)
import jax
import jax.numpy as jnp


def kernel(x, edge_index, edge_attr, emb1, emb2, ee1, ee2, W1, b1, W2, b2,
           eps, bn_g, bn_b):
    n = x.shape[0]
    h = emb1[x[:, 0]] + emb2[x[:, 1]]
    loop = jnp.arange(n, dtype=edge_index.dtype)
    ei = jnp.concatenate([edge_index, jnp.stack([loop, loop], axis=0)], axis=1)
    sl_attr = jnp.concatenate([jnp.full((n, 1), 4, dtype=edge_attr.dtype),
                               jnp.zeros((n, 1), dtype=edge_attr.dtype)], axis=1)
    ea = jnp.concatenate([edge_attr, sl_attr], axis=0)
    order = jnp.argsort(ei[1], stable=True)
    ei = ei[:, order]
    ea = ea[order]
    src = ei[0]
    dst = ei[1]
    for i in range(5):
        eemb = ee1[i][ea[:, 0]] + ee2[i][ea[:, 1]]
        msg = h[src] + eemb
        agg = jax.ops.segment_sum(msg, dst, num_segments=n)
        z = (1.0 + eps[i]) * agg
        z = jnp.maximum(z @ W1[i] + b1[i], 0.0) @ W2[i] + b2[i]
        mu = jnp.mean(z, axis=0)
        var = jnp.var(z, axis=0)
        z = (z - mu) / jnp.sqrt(var + 1e-5) * bn_g[i] + bn_b[i]
        if i < 4:
            z = jnp.maximum(z, 0.0)
        h = z
    return h


# speedup vs baseline: 1.0649x; 1.0649x over previous
import functools

import jax
import jax.numpy as jnp
from jax import lax
from jax.experimental import pallas as pl
from jax.experimental.pallas import tpu as pltpu
from jax.experimental.pallas import tpu_sc as plsc

NC = 2
NS = 16
NW = NC * NS
K = 128
NLOC = 320
BIG = 1 << 20


def _gin_agg_sc(h, idx3, eetab, nch, nself, n_out, d, m_pad):
    mesh = plsc.VectorSubcoreMesh(core_axis_name="c", subcore_axis_name="s")
    racc = NS * NLOC + 1

    @functools.partial(
        pl.kernel,
        out_type=jax.ShapeDtypeStruct((n_out, d), jnp.float32),
        mesh=mesh,
        scratch_types=[
            pltpu.VMEM((3, K), jnp.int32),
            pltpu.VMEM((K, d), jnp.float32),
            pltpu.VMEM((K, d), jnp.float32),
            pltpu.VMEM_SHARED((racc, d), jnp.float32),
            pltpu.SemaphoreType.DMA,
            pltpu.SemaphoreType.DMA,
        ],
    )
    def kern(h_hbm, idx_hbm, ee_hbm, out_hbm, ibuf, bufh, bufe, acc,
             semh, seme):
        c = lax.axis_index("c")
        s = lax.axis_index("s")
        w = s * NC + c
        lo = w * NLOC
        sb = s * NLOC

        zv = jnp.zeros((16,), jnp.float32)

        def zrow(r, carry):
            for j in range(d // 16):
                bufh[r, pl.ds(j * 16, 16)] = zv
            return carry

        lax.fori_loop(0, K, zrow, 0)
        for q in range(NLOC // K):
            pltpu.sync_copy(bufh, acc.at[pl.ds(sb + q * K, K)])

        def do_chunk(ci, always):
            pltpu.sync_copy(idx_hbm.at[ci], ibuf)
            for j in range(K // 16):
                dstv = ibuf[2, pl.ds(j * 16, 16)]
                ok = (dstv >= lo) & (dstv < lo + NLOC)
                ibuf[2, pl.ds(j * 16, 16)] = jnp.where(
                    ok, dstv - (lo - sb), NS * NLOC)

            def work():
                cph = pltpu.async_copy(h_hbm.at[ibuf.at[0]], bufh, semh)
                cpe = pltpu.async_copy(ee_hbm.at[ibuf.at[1]], bufe, seme)
                cph.wait()
                cpe.wait()

                def mrow(e0, carry):
                    for j in range(d // 16):
                        plsc.addupdate(bufh.at[e0, pl.ds(j * 16, 16)],
                                       bufe[e0, pl.ds(j * 16, 16)])
                    return carry

                lax.fori_loop(0, K, mrow, 0)

                pltpu.sync_copy(bufh, acc.at[ibuf.at[2]], add=True)

            work()

        def chunk_body(ci, carry):
            do_chunk(ci, False)
            return carry

        c0 = jnp.maximum(w * nch // NW - m_pad, 0)
        c1 = jnp.minimum((w + 1) * nch // NW + m_pad, nch)
        lax.fori_loop(c0, c1, chunk_body, 0)
        for q in range(nself):
            do_chunk(nch + nself * w + q, True)

        pltpu.sync_copy(acc.at[pl.ds(sb, NLOC)],
                        out_hbm.at[pl.ds(lo, NLOC)])

    return kern(h, idx3, eetab)


def kernel(x, edge_index, edge_attr, emb1, emb2, ee1, ee2, W1, b1, W2, b2,
           eps, bn_g, bn_b):
    n, d = x.shape[0], emb1.shape[1]
    e = edge_index.shape[1]
    nl = W1.shape[0]
    n_out = NW * NLOC

    src = edge_index[0].astype(jnp.int32)
    dst = edge_index[1].astype(jnp.int32)
    t = (edge_attr[:, 0] * 3 + edge_attr[:, 1]).astype(jnp.int32)
    order = jnp.argsort(dst, stable=True)
    src_s, dst_s, t_s = src[order], dst[order], t[order]

    nch = -(-e // K)
    padk = nch * K - e
    if padk:
        src_s = jnp.concatenate([src_s, jnp.zeros((padk,), jnp.int32)])
        t_s = jnp.concatenate([t_s, jnp.zeros((padk,), jnp.int32)])
        dst_s = jnp.concatenate([dst_s, jnp.full((padk,), BIG, jnp.int32)])
    real = jnp.stack([src_s.reshape(nch, K), t_s.reshape(nch, K),
                      dst_s.reshape(nch, K)], axis=1)

    nself = -(-NLOC // K)
    node = jnp.arange(NW * nself * K, dtype=jnp.int32)
    wsl = node // (nself * K)
    islot = node % (nself * K)
    vnode = wsl * NLOC + islot
    valid = (islot < NLOC) & (vnode < n)
    sl_src = jnp.where(valid, jnp.minimum(vnode, n - 1), 0)
    sl_dst = jnp.where(valid, vnode, BIG)
    sl_t = jnp.full_like(node, 12)
    selfc = jnp.stack([sl_src.reshape(NW * nself, K),
                       sl_t.reshape(NW * nself, K),
                       sl_dst.reshape(NW * nself, K)], axis=1)
    idx3 = jnp.concatenate([real, selfc], axis=0)

    wb = NLOC * jnp.arange(NW + 1, dtype=jnp.int32)
    epos = jnp.searchsorted(dst_s[:e], wb)
    c0t = epos[:-1] // K
    c1t = -(-epos[1:] // K)
    ww = jnp.arange(NW)
    m_pad = 16
    c0s = jnp.maximum(ww * nch // NW - m_pad, 0)
    c1s = jnp.minimum((ww + 1) * nch // NW + m_pad, nch)
    covered = jnp.all((c0s <= c0t) & (c1t <= c1s))

    tt = jnp.arange(16)
    EE = ee1[:, jnp.clip(tt // 3, 0, 5)] + ee2[:, tt % 3]

    h0 = emb1[x[:, 0]] + emb2[x[:, 1]]

    def mlp_bn(z, i):
        z = jnp.maximum(z @ W1[i] + b1[i], 0.0) @ W2[i] + b2[i]
        mu = jnp.mean(z, axis=0)
        var = jnp.var(z, axis=0)
        z = (z - mu) / jnp.sqrt(var + 1e-5) * bn_g[i] + bn_b[i]
        return jnp.maximum(z, 0.0) if i < nl - 1 else z

    def fast(h):
        for i in range(nl):
            agg = _gin_agg_sc(h, idx3, EE[i], nch, nself, n_out, d,
                              m_pad)[:n]
            h = mlp_bn((1.0 + eps[i]) * agg, i)
        return h

    def slow(h):
        loop = jnp.arange(n, dtype=jnp.int32)
        ei = jnp.concatenate([jnp.stack([src, dst], 0),
                              jnp.stack([loop, loop], 0)], axis=1)
        ta = jnp.concatenate([t, jnp.full((n,), 12, jnp.int32)])
        for i in range(nl):
            msg = h[ei[0]] + EE[i][ta]
            agg = jax.ops.segment_sum(msg, ei[1], num_segments=n)
            h = mlp_bn((1.0 + eps[i]) * agg, i)
        return h

    return lax.cond(covered, fast, slow, h0)


# speedup vs baseline: 1.0654x; 1.0005x over previous
import functools

import jax
import jax.numpy as jnp
from jax import lax
from jax.experimental import pallas as pl
from jax.experimental.pallas import tpu as pltpu
from jax.experimental.pallas import tpu_sc as plsc

NC = 2
NS = 16
NW = NC * NS
K = 128
NLOC = 320
BIG = 1 << 20


def _gin_agg_sc(h, idx3, eetab, nch, nself, n_out, d, m_pad, cw):
    mesh = plsc.VectorSubcoreMesh(core_axis_name="c", subcore_axis_name="s")
    racc = NS * NLOC + 1

    @functools.partial(
        pl.kernel,
        out_type=jax.ShapeDtypeStruct((n_out, d), jnp.float32),
        mesh=mesh,
        scratch_types=[
            pltpu.VMEM((3, K), jnp.int32),
            pltpu.VMEM((3, K), jnp.int32),
            pltpu.VMEM((K, d), jnp.float32),
            pltpu.VMEM((K, d), jnp.float32),
            pltpu.VMEM((K, d), jnp.float32),
            pltpu.VMEM((K, d), jnp.float32),
            pltpu.VMEM_SHARED((racc, d), jnp.float32),
            pltpu.SemaphoreType.DMA,
            pltpu.SemaphoreType.DMA,
            pltpu.SemaphoreType.DMA,
            pltpu.SemaphoreType.DMA,
        ],
    )
    def kern(h_hbm, idx_hbm, ee_hbm, out_hbm, iba, ibb, bha, bhb, bea, beb,
             acc, sha, shb, sea, seb):
        c = lax.axis_index("c")
        s = lax.axis_index("s")
        w = s * NC + c
        lo = w * NLOC
        sb = s * NLOC

        zv = jnp.zeros((16,), jnp.float32)

        def zrow(r, carry):
            for j in range(d // 16):
                bha[r, pl.ds(j * 16, 16)] = zv
            return carry

        lax.fori_loop(0, K, zrow, 0)
        for q in range(NLOC // K):
            pltpu.sync_copy(bha, acc.at[pl.ds(sb + q * K, K)])

        c0 = jnp.minimum(jnp.maximum(w * nch // NW - m_pad, 0), nch - cw)
        selfbase = nch + nself * w
        cmax = nch + nself * NW - 1

        def cidx(k):
            return jnp.minimum(jnp.where(k < cw, c0 + k,
                                         selfbase + (k - cw)), cmax)

        def stage(ib, k):
            pltpu.sync_copy(idx_hbm.at[cidx(k)], ib)

        def fire(ib, bh, be, sh, se):
            pltpu.async_copy(h_hbm.at[ib.at[0]], bh, sh)
            pltpu.async_copy(ee_hbm.at[ib.at[1]], be, se)

        def proc(ib, bh, be, sh, se):
            pltpu.make_async_copy(h_hbm.at[pl.ds(0, K)], bh, sh).wait()
            pltpu.make_async_copy(ee_hbm.at[pl.ds(0, K)], be, se).wait()

            def mrow(e0, carry):
                for j in range(d // 16):
                    plsc.addupdate(bh.at[e0, pl.ds(j * 16, 16)],
                                   be[e0, pl.ds(j * 16, 16)])
                return carry

            lax.fori_loop(0, K, mrow, 0)
            for j in range(K // 16):
                dstv = ib[2, pl.ds(j * 16, 16)]
                ok = (dstv >= lo) & (dstv < lo + NLOC)
                ib[2, pl.ds(j * 16, 16)] = jnp.where(
                    ok, dstv - (lo - sb), NS * NLOC)
            pltpu.sync_copy(bh, acc.at[ib.at[2]], add=True)

        stage(iba, 0)
        fire(iba, bha, bea, sha, sea)

        def pair(g, carry):
            k0 = 2 * g
            stage(ibb, k0 + 1)
            fire(ibb, bhb, beb, shb, seb)
            proc(iba, bha, bea, sha, sea)
            stage(iba, k0 + 2)
            fire(iba, bha, bea, sha, sea)
            proc(ibb, bhb, beb, shb, seb)
            return carry

        lax.fori_loop(0, (cw + nself) // 2, pair, 0)
        pltpu.make_async_copy(h_hbm.at[pl.ds(0, K)], bha, sha).wait()
        pltpu.make_async_copy(ee_hbm.at[pl.ds(0, K)], bea, sea).wait()

        pltpu.sync_copy(acc.at[pl.ds(sb, NLOC)],
                        out_hbm.at[pl.ds(lo, NLOC)])

    return kern(h, idx3, eetab)


def kernel(x, edge_index, edge_attr, emb1, emb2, ee1, ee2, W1, b1, W2, b2,
           eps, bn_g, bn_b):
    n, d = x.shape[0], emb1.shape[1]
    e = edge_index.shape[1]
    nl = W1.shape[0]
    n_out = NW * NLOC

    src = edge_index[0].astype(jnp.int32)
    dst = edge_index[1].astype(jnp.int32)
    t = (edge_attr[:, 0] * 3 + edge_attr[:, 1]).astype(jnp.int32)
    order = jnp.argsort(dst, stable=True)
    src_s, dst_s, t_s = src[order], dst[order], t[order]

    nch = -(-e // K)
    padk = nch * K - e
    if padk:
        src_s = jnp.concatenate([src_s, jnp.zeros((padk,), jnp.int32)])
        t_s = jnp.concatenate([t_s, jnp.zeros((padk,), jnp.int32)])
        dst_s = jnp.concatenate([dst_s, jnp.full((padk,), BIG, jnp.int32)])
    real = jnp.stack([src_s.reshape(nch, K), t_s.reshape(nch, K),
                      dst_s.reshape(nch, K)], axis=1)

    nself = 4
    node = jnp.arange(NW * nself * K, dtype=jnp.int32)
    wsl = node // (nself * K)
    islot = node % (nself * K)
    vnode = wsl * NLOC + islot
    valid = (islot < NLOC) & (vnode < n)
    sl_src = jnp.where(valid, jnp.minimum(vnode, n - 1), 0)
    sl_dst = jnp.where(valid, vnode, BIG)
    sl_t = jnp.full_like(node, 12)
    selfc = jnp.stack([sl_src.reshape(NW * nself, K),
                       sl_t.reshape(NW * nself, K),
                       sl_dst.reshape(NW * nself, K)], axis=1)
    idx3 = jnp.concatenate([real, selfc], axis=0)

    wb = NLOC * jnp.arange(NW + 1, dtype=jnp.int32)
    epos = jnp.searchsorted(dst_s[:e], wb)
    c0t = epos[:-1] // K
    c1t = -(-epos[1:] // K)
    ww = jnp.arange(NW)
    m_pad = 12
    cw = nch // NW + 2 * m_pad + 2
    if cw % 2:
        cw += 1
    c0s = jnp.clip(ww * nch // NW - m_pad, 0, nch - cw)
    covered = jnp.all((c0s <= c0t) & (c1t <= c0s + cw))

    tt = jnp.arange(16)
    EE = ee1[:, jnp.clip(tt // 3, 0, 5)] + ee2[:, tt % 3]
    EE = jnp.concatenate(
        [EE, jnp.zeros((nl, K - 16, d), jnp.float32)], axis=1)

    h0 = emb1[x[:, 0]] + emb2[x[:, 1]]

    def mlp_bn(z, i):
        z = jnp.maximum(z @ W1[i] + b1[i], 0.0) @ W2[i] + b2[i]
        mu = jnp.mean(z, axis=0)
        var = jnp.var(z, axis=0)
        z = (z - mu) / jnp.sqrt(var + 1e-5) * bn_g[i] + bn_b[i]
        return jnp.maximum(z, 0.0) if i < nl - 1 else z

    def fast(h):
        for i in range(nl):
            agg = _gin_agg_sc(h, idx3, EE[i], nch, nself, n_out, d,
                              m_pad, cw)[:n]
            h = mlp_bn((1.0 + eps[i]) * agg, i)
        return h

    def slow(h):
        loop = jnp.arange(n, dtype=jnp.int32)
        ei = jnp.concatenate([jnp.stack([src, dst], 0),
                              jnp.stack([loop, loop], 0)], axis=1)
        ta = jnp.concatenate([t, jnp.full((n,), 12, jnp.int32)])
        for i in range(nl):
            msg = h[ei[0]] + EE[i][ta]
            agg = jax.ops.segment_sum(msg, ei[1], num_segments=n)
            h = mlp_bn((1.0 + eps[i]) * agg, i)
        return h

    return lax.cond(covered, fast, slow, h0)
